# baseline (speedup 1.0000x reference)
import functools

import jax
import jax.numpy as jnp
from jax import lax
from jax.experimental import pallas as pl
from jax.experimental.pallas import tpu as pltpu
from jax.experimental.pallas import tpu_sc as plsc

NUM_CLS = 1000
B = 16384
LANES = 16
NUM_CORES = 2
NUM_SUBCORES = 16
NW = NUM_CORES * NUM_SUBCORES
COLS_PER_W = B // NW
CLS_CHUNK = 120
COL_GROUPS = COLS_PER_W // LANES

_BANDS = [(NUM_CLS - NUM_CLS % CLS_CHUNK, NUM_CLS % CLS_CHUNK)]
_BANDS += [(k * CLS_CHUNK, CLS_CHUNK) for k in range(NUM_CLS // CLS_CHUNK)]


def _sc_body(x_hbm, out_hbm, idx_v, buf0, buf1, sem0, sem1):
    cid = lax.axis_index("c")
    sid = lax.axis_index("s")
    wid = sid * NUM_CORES + cid
    cbase = wid * COLS_PER_W

    pltpu.sync_copy(x_hbm.at[pl.ds(cbase, COLS_PER_W)], idx_v)

    zeros16 = jnp.zeros((LANES,), jnp.float32)
    ones16 = jnp.ones((LANES,), jnp.float32)
    lane = lax.iota(jnp.int32, LANES)

    def zero_rows(buf, r0, r1):
        def zero_row(r, _):
            for g in range(COL_GROUPS):
                buf[r, pl.ds(g * LANES, LANES)] = zeros16
            return ()

        lax.fori_loop(r0, r1, zero_row, ())

    def scan_band(lo, hi, buf):
        def grp(g, _):
            xv = idx_v[pl.ds(g * LANES, LANES)]
            m = (xv >= lo) & (xv < hi)
            rows = jnp.where(m, xv - lo, 0)
            cols = lane + g * LANES
            plsc.store_scatter(buf, [rows, cols], ones16, mask=m)
            return ()

        lax.fori_loop(0, COL_GROUPS, grp, ())

    def unset_set_band(plo, phi, lo, hi, buf):
        def grp(g, _):
            xv = idx_v[pl.ds(g * LANES, LANES)]
            cols = lane + g * LANES
            m0 = (xv >= plo) & (xv < phi)
            rows0 = jnp.where(m0, xv - plo, 0)
            plsc.store_scatter(buf, [rows0, cols], zeros16, mask=m0)
            m1 = (xv >= lo) & (xv < hi)
            rows1 = jnp.where(m1, xv - lo, 0)
            plsc.store_scatter(buf, [rows1, cols], ones16, mask=m1)
            return ()

        lax.fori_loop(0, COL_GROUPS, grp, ())

    bufs = (buf0, buf1)
    sems = (sem0, sem1)
    pending = [None, None]
    for k, (lo, rows) in enumerate(_BANDS):
        b = k % 2
        buf = bufs[b]
        if k < 2:
            zero_rows(buf, 0, rows if k == 0 else CLS_CHUNK)
            scan_band(lo, lo + rows, buf)
        else:
            pending[b].wait()
            plo, prows = _BANDS[k - 2]
            unset_set_band(plo, plo + prows, lo, lo + rows, buf)
        cp = pltpu.make_async_copy(
            buf.at[pl.ds(0, rows)],
            out_hbm.at[pl.ds(lo, rows), pl.ds(cbase, COLS_PER_W)],
            sems[b],
        )
        cp.start()
        pending[b] = cp
        if k == 0 and rows < CLS_CHUNK:
            zero_rows(buf, rows, CLS_CHUNK)
    pending[0].wait()
    pending[1].wait()


_mesh = plsc.VectorSubcoreMesh(core_axis_name="c", subcore_axis_name="s")

_sc_call = functools.partial(
    pl.kernel,
    out_type=jax.ShapeDtypeStruct((NUM_CLS, B), jnp.float32),
    mesh=_mesh,
    compiler_params=pltpu.CompilerParams(
        use_tc_tiling_on_sc=True, needs_layout_passes=False
    ),
    scratch_types=[
        pltpu.VMEM((COLS_PER_W,), jnp.int32),
        pltpu.VMEM((CLS_CHUNK, COLS_PER_W), jnp.float32),
        pltpu.VMEM((CLS_CHUNK, COLS_PER_W), jnp.float32),
        pltpu.SemaphoreType.DMA,
        pltpu.SemaphoreType.DMA,
    ],
)(_sc_body)


def kernel(x):
    return _sc_call(x).T


# speedup vs baseline: 1.0399x; 1.0399x over previous
import functools

import jax
import jax.numpy as jnp
from jax import lax
from jax.experimental import pallas as pl
from jax.experimental.pallas import tpu as pltpu
from jax.experimental.pallas import tpu_sc as plsc

NUM_CLS = 1000
B = 16384
LANES = 16
NUM_CORES = 2
NUM_SUBCORES = 16
NW = NUM_CORES * NUM_SUBCORES
COLS_PER_W = B // NW
CLS_CHUNK = 120
COL_GROUPS = COLS_PER_W // LANES

_BANDS = [(k * CLS_CHUNK, CLS_CHUNK) for k in range(NUM_CLS // CLS_CHUNK)]
_BANDS.append((NUM_CLS - NUM_CLS % CLS_CHUNK, NUM_CLS % CLS_CHUNK))


def _sc_body(x_hbm, out_hbm, idx_v, buf0, buf1, sem0, sem1):
    cid = lax.axis_index("c")
    sid = lax.axis_index("s")
    wid = sid * NUM_CORES + cid
    cbase = wid * COLS_PER_W

    pltpu.sync_copy(x_hbm.at[pl.ds(cbase, COLS_PER_W)], idx_v)

    zeros16 = jnp.zeros((LANES,), jnp.float32)
    ones16 = jnp.ones((LANES,), jnp.float32)
    lane = lax.iota(jnp.int32, LANES)

    def zero_rows(buf, r0, r1):
        def zero_row(r, _):
            for g in range(COL_GROUPS):
                buf[r, pl.ds(g * LANES, LANES)] = zeros16
            return ()

        lax.fori_loop(r0, r1, zero_row, ())

    def scan_band(lo, hi, buf):
        def grp(g, _):
            xv = idx_v[pl.ds(g * LANES, LANES)]
            m = (xv >= lo) & (xv < hi)
            rows = jnp.where(m, xv - lo, 0)
            cols = lane + g * LANES
            plsc.store_scatter(buf, [rows, cols], ones16, mask=m)
            return ()

        lax.fori_loop(0, COL_GROUPS, grp, ())

    def unset_set_band(plo, phi, lo, hi, buf):
        def grp(g, _):
            xv = idx_v[pl.ds(g * LANES, LANES)]
            cols = lane + g * LANES
            m0 = (xv >= plo) & (xv < phi)
            rows0 = jnp.where(m0, xv - plo, 0)
            plsc.store_scatter(buf, [rows0, cols], zeros16, mask=m0)
            m1 = (xv >= lo) & (xv < hi)
            rows1 = jnp.where(m1, xv - lo, 0)
            plsc.store_scatter(buf, [rows1, cols], ones16, mask=m1)
            return ()

        lax.fori_loop(0, COL_GROUPS, grp, ())

    bufs = (buf0, buf1)
    sems = (sem0, sem1)
    pending = [None, None]
    for k, (lo, rows) in enumerate(_BANDS):
        b = k % 2
        buf = bufs[b]
        if k < 2:
            zero_rows(buf, 0, CLS_CHUNK)
            scan_band(lo, lo + rows, buf)
        else:
            pending[b].wait()
            plo, prows = _BANDS[k - 2]
            unset_set_band(plo, plo + prows, lo, lo + rows, buf)
        cp = pltpu.make_async_copy(
            buf.at[pl.ds(0, rows)],
            out_hbm.at[pl.ds(lo, rows), pl.ds(cbase, COLS_PER_W)],
            sems[b],
        )
        cp.start()
        pending[b] = cp
    pending[0].wait()
    pending[1].wait()


_mesh = plsc.VectorSubcoreMesh(core_axis_name="c", subcore_axis_name="s")

_sc_call = functools.partial(
    pl.kernel,
    out_type=jax.ShapeDtypeStruct((NUM_CLS, B), jnp.float32),
    mesh=_mesh,
    compiler_params=pltpu.CompilerParams(
        use_tc_tiling_on_sc=True, needs_layout_passes=False
    ),
    scratch_types=[
        pltpu.VMEM((COLS_PER_W,), jnp.int32),
        pltpu.VMEM((CLS_CHUNK, COLS_PER_W), jnp.float32),
        pltpu.VMEM((CLS_CHUNK, COLS_PER_W), jnp.float32),
        pltpu.SemaphoreType.DMA,
        pltpu.SemaphoreType.DMA,
    ],
)(_sc_body)


def kernel(x):
    return _sc_call(x).T


# speedup vs baseline: 1.0452x; 1.0051x over previous
import functools

import jax
import jax.numpy as jnp
from jax import lax
from jax.experimental import pallas as pl
from jax.experimental.pallas import tpu as pltpu
from jax.experimental.pallas import tpu_sc as plsc

NUM_CLS = 1000
B = 16384
LANES = 16
NUM_CORES = 2
NUM_SUBCORES = 16
NW = NUM_CORES * NUM_SUBCORES
COLS_PER_W = B // NW
CLS_CHUNK = 80
NBUF = 3
COL_GROUPS = COLS_PER_W // LANES

_BANDS = [(k * CLS_CHUNK, CLS_CHUNK) for k in range(NUM_CLS // CLS_CHUNK)]
_BANDS.append((NUM_CLS - NUM_CLS % CLS_CHUNK, NUM_CLS % CLS_CHUNK))


def _sc_body(x_hbm, out_hbm, idx_v, buf0, buf1, buf2, sem0, sem1, sem2):
    cid = lax.axis_index("c")
    sid = lax.axis_index("s")
    wid = sid * NUM_CORES + cid
    cbase = wid * COLS_PER_W

    pltpu.sync_copy(x_hbm.at[pl.ds(cbase, COLS_PER_W)], idx_v)

    zeros16 = jnp.zeros((LANES,), jnp.float32)
    ones16 = jnp.ones((LANES,), jnp.float32)
    lane = lax.iota(jnp.int32, LANES)

    def zero_buf(buf):
        def zero_row(r, _):
            for g in range(COL_GROUPS):
                buf[r, pl.ds(g * LANES, LANES)] = zeros16
            return ()

        lax.fori_loop(0, CLS_CHUNK, zero_row, ())

    def scan_band(lo, hi, buf):
        def grp(g, _):
            xv = idx_v[pl.ds(g * LANES, LANES)]
            m = (xv >= lo) & (xv < hi)
            rows = jnp.where(m, xv - lo, 0)
            cols = lane + g * LANES
            plsc.store_scatter(buf, [rows, cols], ones16, mask=m)
            return ()

        lax.fori_loop(0, COL_GROUPS, grp, ())

    def unset_set_band(plo, phi, lo, hi, buf):
        def grp(g, _):
            xv = idx_v[pl.ds(g * LANES, LANES)]
            cols = lane + g * LANES
            m0 = (xv >= plo) & (xv < phi)
            rows0 = jnp.where(m0, xv - plo, 0)
            plsc.store_scatter(buf, [rows0, cols], zeros16, mask=m0)
            m1 = (xv >= lo) & (xv < hi)
            rows1 = jnp.where(m1, xv - lo, 0)
            plsc.store_scatter(buf, [rows1, cols], ones16, mask=m1)
            return ()

        lax.fori_loop(0, COL_GROUPS, grp, ())

    bufs = (buf0, buf1, buf2)
    sems = (sem0, sem1, sem2)
    pending = [None, None, None]
    for k, (lo, rows) in enumerate(_BANDS):
        b = k % NBUF
        buf = bufs[b]
        if k < NBUF:
            zero_buf(buf)
            scan_band(lo, lo + rows, buf)
        else:
            pending[b].wait()
            plo, prows = _BANDS[k - NBUF]
            unset_set_band(plo, plo + prows, lo, lo + rows, buf)
        cp = pltpu.make_async_copy(
            buf.at[pl.ds(0, rows)],
            out_hbm.at[pl.ds(lo, rows), pl.ds(cbase, COLS_PER_W)],
            sems[b],
        )
        cp.start()
        pending[b] = cp
    for cp in pending:
        cp.wait()


_mesh = plsc.VectorSubcoreMesh(core_axis_name="c", subcore_axis_name="s")

_sc_call = functools.partial(
    pl.kernel,
    out_type=jax.ShapeDtypeStruct((NUM_CLS, B), jnp.float32),
    mesh=_mesh,
    compiler_params=pltpu.CompilerParams(
        use_tc_tiling_on_sc=True, needs_layout_passes=False
    ),
    scratch_types=[
        pltpu.VMEM((COLS_PER_W,), jnp.int32),
        pltpu.VMEM((CLS_CHUNK, COLS_PER_W), jnp.float32),
        pltpu.VMEM((CLS_CHUNK, COLS_PER_W), jnp.float32),
        pltpu.VMEM((CLS_CHUNK, COLS_PER_W), jnp.float32),
        pltpu.SemaphoreType.DMA,
        pltpu.SemaphoreType.DMA,
        pltpu.SemaphoreType.DMA,
    ],
)(_sc_body)


def kernel(x):
    return _sc_call(x).T
